# baseline (speedup 1.0000x reference)
import jax
import jax.numpy as jnp
from jax import lax
from jax.experimental import pallas as pl
from jax.experimental.pallas import tpu as pltpu
from jax.experimental.pallas import tpu_sc as plsc

_F = 26
_V = 100000
_E = 16
_B = 4096
_BCONST = 0.99

_NC = 2
_NS = 16
_NW = _NC * _NS
_BPW = _B // _NW
_RPW = _BPW * _F // 128
_NG = _BPW // 16

_M = _V // 8
_VC = 12800
_MR = _VC // 8
_NVB = 8
_FB = (_V + 127) // 128


def _tr_body(x_ref, o_ref):
    x = x_ref[0]
    r = lax.broadcasted_iota(jnp.int32, (_E, _E), 0)
    c = lax.broadcasted_iota(jnp.int32, (_E, _E), 1)
    eye = (r == c).astype(jnp.float32)
    y = lax.dot_general(x, eye, (((0,), (0,)), ((), ())),
                        preferred_element_type=jnp.float32)
    rr = y.reshape(_MR, 8, _E)
    o_ref[0] = jnp.concatenate([rr[:, s, :] for s in range(8)],
                               axis=1)


def _tc_repack(sw_t):
    return pl.pallas_call(
        _tr_body,
        grid=(_F, _NVB),
        in_specs=[pl.BlockSpec((1, _E, _VC), lambda f, c: (f, 0, c))],
        out_specs=pl.BlockSpec((1, _MR, 128), lambda f, c: (f, c, 0)),
        out_shape=jax.ShapeDtypeStruct((_F, _M, 128), jnp.float32),
    )(sw_t)


def _fm_body(xi_h, xv_h, fwc_h, swc_h, out_h,
             xi_v, xv_v, midx_v, fidx_v, sbuf, fbuf,
             sacc, qacc, foacc, out_v, sem):
    wid = lax.axis_index("s") * _NC + lax.axis_index("c")
    pltpu.sync_copy(xi_h.at[wid], xi_v)
    pltpu.sync_copy(xv_h.at[wid], xv_v)

    lane = lax.iota(jnp.int32, 16)
    zero = jnp.zeros((16,), jnp.float32)

    def idx_body(k, carry):
        f = k // _NG
        g = k % _NG
        gs = pl.ds(g * 16, 16)
        v = xi_v[f, gs]
        midx_v[f, gs] = lax.shift_right_logical(v, 3)
        fidx_v[f, gs] = lax.shift_right_logical(v, 7)
        return carry
    lax.fori_loop(0, _F * _NG, idx_body, 0)

    def z_body(g, carry):
        gs = pl.ds(g * 16, 16)
        qacc[0, gs] = zero
        foacc[0, gs] = zero
        for e in range(_E):
            sacc[e, gs] = zero
        return carry
    lax.fori_loop(0, _NG, z_body, 0)

    for f in range(_F):
        cs = pltpu.async_copy(swc_h.at[f].at[midx_v.at[f]], sbuf, sem)
        cf = pltpu.async_copy(fwc_h.at[f].at[fidx_v.at[f]], fbuf, sem)
        cs.wait()
        cf.wait()

        def c_body(g, carry, f=f):
            gs = pl.ds(g * 16, 16)
            v16 = xi_v[f, gs]
            rows = g * 16 + lane
            scol = (v16 & 7) * 16
            fcol = v16 & 127
            xvb = xv_v[f, gs]
            fval = plsc.load_gather(fbuf, [rows, fcol])
            foacc[0, gs] = foacc[0, gs] + fval * xvb
            q = qacc[0, gs]
            for e in range(_E):
                sval = plsc.load_gather(sbuf, [rows, scol + e])
                vv = sval * xvb
                sacc[e, gs] = sacc[e, gs] + vv
                q = q + vv * vv
            qacc[0, gs] = q
            return carry
        lax.fori_loop(0, _NG, c_body, 0)

    def o_body(g, carry):
        gs = pl.ds(g * 16, 16)
        s2 = zero
        for e in range(_E):
            se = sacc[e, gs]
            s2 = s2 + se * se
        out_v[gs] = _BCONST + foacc[0, gs] + 0.5 * (s2 - qacc[0, gs])
        return carry
    lax.fori_loop(0, _NG, o_body, 0)

    pltpu.sync_copy(out_v, out_h.at[pl.ds(wid * _BPW, _BPW)])


def _fm_call(xi3, xv3, fwc, swc):
    mesh = plsc.VectorSubcoreMesh(core_axis_name="c", subcore_axis_name="s")
    return pl.kernel(
        _fm_body,
        out_type=jax.ShapeDtypeStruct((_B,), jnp.float32),
        mesh=mesh,
        scratch_types=[
            pltpu.VMEM((_F, _BPW), jnp.int32),
            pltpu.VMEM((_F, _BPW), jnp.float32),
            pltpu.VMEM((_F, _BPW), jnp.int32),
            pltpu.VMEM((_F, _BPW), jnp.int32),
            pltpu.VMEM((_BPW, 128), jnp.float32),
            pltpu.VMEM((_BPW, 128), jnp.float32),
            pltpu.VMEM((_E, _BPW), jnp.float32),
            pltpu.VMEM((1, _BPW), jnp.float32),
            pltpu.VMEM((1, _BPW), jnp.float32),
            pltpu.VMEM((_BPW,), jnp.float32),
            pltpu.SemaphoreType.DMA,
        ],
        compiler_params=pltpu.CompilerParams(needs_layout_passes=False,
                                             use_tc_tiling_on_sc=True),
    )(xi3, xv3, fwc, swc)


@jax.jit
def kernel(Xi, Xv, first_w, second_w):
    swc = second_w.reshape(_F, _M, 128)
    fwc = jnp.pad(first_w.reshape(_F, _V),
                  ((0, 0), (0, _FB * 128 - _V))).reshape(_F, _FB, 128)
    xi3 = (Xi.reshape(_B, _F).astype(jnp.int32).T
           .reshape(_F, _NW, _BPW).transpose(1, 0, 2))
    xv3 = Xv.T.reshape(_F, _NW, _BPW).transpose(1, 0, 2)
    return _fm_call(xi3, xv3, fwc, swc)


# speedup vs baseline: 1.3303x; 1.3303x over previous
import jax
import jax.numpy as jnp
from jax import lax
from jax.experimental import pallas as pl
from jax.experimental.pallas import tpu as pltpu
from jax.experimental.pallas import tpu_sc as plsc

_F = 26
_V = 100000
_E = 16
_B = 4096
_BCONST = 0.99

_NC = 2
_NS = 16
_NW = _NC * _NS
_BPW = _B // _NW
_RPW = _BPW * _F // 128
_NG = _BPW // 16

_M = _V // 8
_VC = 12800
_MR = _VC // 8
_NVB = 8
_FB = (_V + 127) // 128


def _tr_body(x_ref, o_ref):
    x = x_ref[0]
    r = lax.broadcasted_iota(jnp.int32, (_E, _E), 0)
    c = lax.broadcasted_iota(jnp.int32, (_E, _E), 1)
    eye = (r == c).astype(jnp.float32)
    y = lax.dot_general(x, eye, (((0,), (0,)), ((), ())),
                        preferred_element_type=jnp.float32)
    rr = y.reshape(_MR, 8, _E)
    o_ref[0] = jnp.concatenate([rr[:, s, :] for s in range(8)],
                               axis=1)


def _tc_repack(sw_t):
    return pl.pallas_call(
        _tr_body,
        grid=(_F, _NVB),
        in_specs=[pl.BlockSpec((1, _E, _VC), lambda f, c: (f, 0, c))],
        out_specs=pl.BlockSpec((1, _MR, 128), lambda f, c: (f, c, 0)),
        out_shape=jax.ShapeDtypeStruct((_F, _M, 128), jnp.float32),
    )(sw_t)


def _fm_body(xi_h, xv_h, fwc_h, swc_h, out_h,
             xi_v, xv_v, midx_v, fidx_v, sbuf, fbuf,
             sacc, qacc, foacc, out_v, sem):
    wid = lax.axis_index("s") * _NC + lax.axis_index("c")
    pltpu.sync_copy(xi_h.at[wid], xi_v)
    pltpu.sync_copy(xv_h.at[wid], xv_v)

    lane = lax.iota(jnp.int32, 16)
    zero = jnp.zeros((16,), jnp.float32)

    def idx_body(k, carry):
        f = k // _NG
        g = k % _NG
        gs = pl.ds(g * 16, 16)
        v = xi_v[f, gs]
        midx_v[f, gs] = lax.shift_right_logical(v, 3)
        fidx_v[f, gs] = lax.shift_right_logical(v, 7)
        return carry
    lax.fori_loop(0, _F * _NG, idx_body, 0)

    def z_body(g, carry):
        gs = pl.ds(g * 16, 16)
        qacc[0, gs] = zero
        foacc[0, gs] = zero
        for e in range(_E):
            sacc[e, gs] = zero
        return carry
    lax.fori_loop(0, _NG, z_body, 0)

    for f in range(_F):
        cs = pltpu.async_copy(swc_h.at[f].at[midx_v.at[f]], sbuf, sem)
        cf = pltpu.async_copy(fwc_h.at[f].at[fidx_v.at[f]], fbuf, sem)
        cs.wait()
        cf.wait()

        def c_body(g, carry, f=f):
            gs = pl.ds(g * 16, 16)
            v16 = xi_v[f, gs]
            rows = g * 16 + lane
            scol = (v16 & 7) * 16
            fcol = v16 & 127
            xvb = xv_v[f, gs]
            fval = plsc.load_gather(fbuf, [rows, fcol])
            foacc[0, gs] = foacc[0, gs] + fval * xvb
            q = qacc[0, gs]
            for e in range(_E):
                sval = plsc.load_gather(sbuf, [rows, scol + e])
                vv = sval * xvb
                sacc[e, gs] = sacc[e, gs] + vv
                q = q + vv * vv
            qacc[0, gs] = q
            return carry
        lax.fori_loop(0, _NG, c_body, 0)

    def o_body(g, carry):
        gs = pl.ds(g * 16, 16)
        s2 = zero
        for e in range(_E):
            se = sacc[e, gs]
            s2 = s2 + se * se
        out_v[gs] = _BCONST + foacc[0, gs] + 0.5 * (s2 - qacc[0, gs])
        return carry
    lax.fori_loop(0, _NG, o_body, 0)

    pltpu.sync_copy(out_v, out_h.at[pl.ds(wid * _BPW, _BPW)])


def _fm_call(xi3, xv3, fwc, swc):
    mesh = plsc.VectorSubcoreMesh(core_axis_name="c", subcore_axis_name="s")
    return pl.kernel(
        _fm_body,
        out_type=jax.ShapeDtypeStruct((_B,), jnp.float32),
        mesh=mesh,
        scratch_types=[
            pltpu.VMEM((_F, _BPW), jnp.int32),
            pltpu.VMEM((_F, _BPW), jnp.float32),
            pltpu.VMEM((_F, _BPW), jnp.int32),
            pltpu.VMEM((_F, _BPW), jnp.int32),
            pltpu.VMEM((_BPW, 128), jnp.float32),
            pltpu.VMEM((_BPW, 128), jnp.float32),
            pltpu.VMEM((_E, _BPW), jnp.float32),
            pltpu.VMEM((1, _BPW), jnp.float32),
            pltpu.VMEM((1, _BPW), jnp.float32),
            pltpu.VMEM((_BPW,), jnp.float32),
            pltpu.SemaphoreType.DMA,
        ],
        compiler_params=pltpu.CompilerParams(needs_layout_passes=False,
                                             use_tc_tiling_on_sc=True),
    )(xi3, xv3, fwc, swc)


@jax.jit
def kernel(Xi, Xv, first_w, second_w):
    swc = _tc_repack(jnp.transpose(second_w, (0, 2, 1)))
    fwc = jnp.pad(first_w.reshape(_F, _V),
                  ((0, 0), (0, _FB * 128 - _V))).reshape(_F, _FB, 128)
    xi3 = (Xi.reshape(_B, _F).astype(jnp.int32).T
           .reshape(_F, _NW, _BPW).transpose(1, 0, 2))
    xv3 = Xv.T.reshape(_F, _NW, _BPW).transpose(1, 0, 2)
    return _fm_call(xi3, xv3, fwc, swc)


# speedup vs baseline: 5.5800x; 4.1945x over previous
import jax
import jax.numpy as jnp
from jax import lax
from jax.experimental import pallas as pl
from jax.experimental.pallas import tpu as pltpu
from jax.experimental.pallas import tpu_sc as plsc

_F = 26
_V = 100000
_E = 16
_B = 4096
_BCONST = 0.99

_NC = 2
_NS = 16
_NW = _NC * _NS
_BPW = _B // _NW
_NG = _BPW // 16

_VC = 102400
_VC8 = _VC // 8
_NVB = -(-_V // _VC)
_M = _NVB * _VC8
_FB = (_V + 127) // 128


def _tr_body(x_ref, o_ref):
    x = x_ref[0]
    xcat = jnp.concatenate([x[:, q * _VC8:(q + 1) * _VC8] for q in range(8)],
                           axis=0)
    r = lax.broadcasted_iota(jnp.int32, (128, 128), 0)
    c = lax.broadcasted_iota(jnp.int32, (128, 128), 1)
    eye = (r == c).astype(jnp.float32)
    o_ref[0] = lax.dot_general(xcat, eye, (((0,), (0,)), ((), ())),
                               preferred_element_type=jnp.float32)


def _tc_repack(sw_t):
    return pl.pallas_call(
        _tr_body,
        grid=(_F, _NVB),
        in_specs=[pl.BlockSpec((1, _E, _VC), lambda f, c: (f, 0, c))],
        out_specs=pl.BlockSpec((1, _VC8, 128), lambda f, c: (f, c, 0)),
        out_shape=jax.ShapeDtypeStruct((_F, _M, 128), jnp.float32),
    )(sw_t)


def _fm_body(xi_h, xv_h, fwc_h, swc_h, out_h,
             xi_v, xv_v, midx_v, fidx_v, sbuf, fbuf,
             sacc, qacc, foacc, out_v, sems, semf):
    wid = lax.axis_index("s") * _NC + lax.axis_index("c")
    pltpu.sync_copy(xi_h.at[wid], xi_v)
    pltpu.sync_copy(xv_h.at[wid], xv_v)

    lane = lax.iota(jnp.int32, 16)
    zero = jnp.zeros((16,), jnp.float32)

    def idx_body(k, carry):
        f = k // _NG
        g = k % _NG
        gs = pl.ds(g * 16, 16)
        v = xi_v[f, gs]
        midx_v[f, gs] = (v // _VC) * _VC8 + v % _VC8
        fidx_v[f, gs] = lax.shift_right_logical(v, 7)
        return carry
    lax.fori_loop(0, _F * _NG, idx_body, 0)

    def z_body(g, carry):
        gs = pl.ds(g * 16, 16)
        qacc[0, gs] = zero
        foacc[0, gs] = zero
        for e in range(_E):
            sacc[e, gs] = zero
        return carry
    lax.fori_loop(0, _NG, z_body, 0)

    def fire(f):
        p = f % 3
        return (pltpu.async_copy(swc_h.at[f].at[midx_v.at[f]],
                                 sbuf.at[p], sems.at[p]),
                pltpu.async_copy(fwc_h.at[f].at[fidx_v.at[f]],
                                 fbuf.at[p], semf.at[p]))

    pend = {0: fire(0), 1: fire(1)}
    for f in range(_F):
        p = f % 3
        if f + 2 < _F:
            pend[f + 2] = fire(f + 2)
        cs, cf = pend.pop(f)
        cs.wait()
        cf.wait()

        def c_body(g, carry, f=f, p=p):
            gs = pl.ds(g * 16, 16)
            v16 = xi_v[f, gs]
            rows = g * 16 + lane
            scol = ((v16 // _VC8) % 8) * 16
            fcol = v16 & 127
            xvb = xv_v[f, gs]
            fval = plsc.load_gather(fbuf.at[p], [rows, fcol])
            foacc[0, gs] = foacc[0, gs] + fval * xvb
            q = qacc[0, gs]
            for e in range(_E):
                sval = plsc.load_gather(sbuf.at[p], [rows, scol + e])
                vv = sval * xvb
                sacc[e, gs] = sacc[e, gs] + vv
                q = q + vv * vv
            qacc[0, gs] = q
            return carry
        lax.fori_loop(0, _NG, c_body, 0)

    def o_body(g, carry):
        gs = pl.ds(g * 16, 16)
        s2 = zero
        for e in range(_E):
            se = sacc[e, gs]
            s2 = s2 + se * se
        out_v[gs] = _BCONST + foacc[0, gs] + 0.5 * (s2 - qacc[0, gs])
        return carry
    lax.fori_loop(0, _NG, o_body, 0)

    pltpu.sync_copy(out_v, out_h.at[pl.ds(wid * _BPW, _BPW)])


def _fm_call(xi3, xv3, fwc, swc):
    mesh = plsc.VectorSubcoreMesh(core_axis_name="c", subcore_axis_name="s")
    return pl.kernel(
        _fm_body,
        out_type=jax.ShapeDtypeStruct((_B,), jnp.float32),
        mesh=mesh,
        scratch_types=[
            pltpu.VMEM((_F, _BPW), jnp.int32),
            pltpu.VMEM((_F, _BPW), jnp.float32),
            pltpu.VMEM((_F, _BPW), jnp.int32),
            pltpu.VMEM((_F, _BPW), jnp.int32),
            pltpu.VMEM((3, _BPW, 128), jnp.float32),
            pltpu.VMEM((3, _BPW, 128), jnp.float32),
            pltpu.VMEM((_E, _BPW), jnp.float32),
            pltpu.VMEM((1, _BPW), jnp.float32),
            pltpu.VMEM((1, _BPW), jnp.float32),
            pltpu.VMEM((_BPW,), jnp.float32),
            pltpu.SemaphoreType.DMA((3,)),
            pltpu.SemaphoreType.DMA((3,)),
        ],
        compiler_params=pltpu.CompilerParams(needs_layout_passes=False,
                                             use_tc_tiling_on_sc=True),
    )(xi3, xv3, fwc, swc)


@jax.jit
def kernel(Xi, Xv, first_w, second_w):
    swc = _tc_repack(jnp.transpose(second_w, (0, 2, 1)))
    fwc = jnp.pad(first_w.reshape(_F, _V),
                  ((0, 0), (0, _FB * 128 - _V))).reshape(_F, _FB, 128)
    xi3 = (Xi.reshape(_B, _F).astype(jnp.int32).T
           .reshape(_F, _NW, _BPW).transpose(1, 0, 2))
    xv3 = Xv.T.reshape(_F, _NW, _BPW).transpose(1, 0, 2)
    return _fm_call(xi3, xv3, fwc, swc)
